# baseline (speedup 1.0000x reference)
import functools

import jax
import jax.numpy as jnp
from jax import lax
from jax.experimental import pallas as pl
from jax.experimental.pallas import tpu as pltpu
from jax.experimental.pallas import tpu_sc as plsc

E = 8
K = 2
T = 2048
D = 768
F = 1024
BM = 256
NT = (T * K + E * (BM - 1)) // BM
PAD_N = NT * BM

_NW = 32
GN = 6144
_CH = 64
_RPW = GN // _NW
_TPW = T // _NW
_VL = 16


def _gating_body(x_ref, gw_ref, gb_ref, logits_ref, idx_ref, w_ref, cnt_ref):
    x = x_ref[...]
    logits = jnp.dot(x, gw_ref[...].T, preferred_element_type=jnp.float32)
    logits = logits + gb_ref[...]
    logits_ref[...] = logits

    eio = lax.broadcasted_iota(jnp.int32, (T, E), 1)
    m1 = jnp.max(logits, axis=1, keepdims=True)
    i1 = jnp.min(jnp.where(logits == m1, eio, E), axis=1, keepdims=True)
    masked = jnp.where(eio == i1, -jnp.inf, logits)
    m2 = jnp.max(masked, axis=1, keepdims=True)
    i2 = jnp.min(jnp.where(masked == m2, eio, E), axis=1, keepdims=True)

    t = jnp.exp(m2 - m1)
    s = 1.0 + t
    w1 = 1.0 / s
    w2 = t / s

    idx_ref[...] = jnp.concatenate([i1, i2], axis=1).astype(jnp.int32)
    w_ref[...] = jnp.concatenate([w1, w2], axis=1)
    cnt1 = (eio == i1).astype(jnp.float32) + (eio == i2).astype(jnp.float32)
    cnt = jnp.sum(cnt1, axis=0, keepdims=True)
    cnt_ref[...] = jnp.concatenate([cnt, jnp.zeros((1, 8), jnp.float32)], axis=1)


def _gating_call(x2, gate_W, gate_b):
    return pl.pallas_call(
        _gating_body,
        out_shape=[
            jax.ShapeDtypeStruct((T, E), jnp.float32),
            jax.ShapeDtypeStruct((T, K), jnp.int32),
            jax.ShapeDtypeStruct((T, K), jnp.float32),
            jax.ShapeDtypeStruct((1, 16), jnp.float32),
        ],
    )(x2, gate_W, gate_b.reshape(1, E))


_A = T * K
_APW = _A // _NW


def _sc_route_body(
    idx_hbm, w_hbm, cnt_hbm,
    p0_hbm, p1_hbm, gid_hbm, ws_hbm, aux_hbm, te_hbm,
    idx_all, wbuf, posb, idsb, p0b, p1b, sv, auxb, teb, cntv, s0, s1,
):
    wid = lax.axis_index("s") * 2 + lax.axis_index("c")
    abase = wid * _APW
    pltpu.sync_copy(idx_hbm, idx_all)
    pltpu.sync_copy(w_hbm.at[pl.ds(abase, _APW)], wbuf)
    pltpu.sync_copy(cnt_hbm.at[0], cntv)

    si = lax.iota(jnp.int32, 16)

    def pre(j, h):
        v = idx_all[pl.ds(j * 16, 16)]
        for e in range(E):
            c = jnp.sum(jnp.where(v == e, 1, 0))
            h = h + jnp.where(si == e, c, 0)
        return h

    hist_v = lax.fori_loop(
        0, wid * (_APW // 16), pre, jnp.zeros((16,), jnp.int32)
    )

    cnt_i = cntv[...].astype(jnp.int32)
    acnt = ((cnt_i + (BM - 1)) // BM) * BM
    aend_al = plsc.cumsum(acnt)
    aoff = aend_al - acnt
    start = aoff + hist_v

    @pl.when(wid == 0)
    def _aux():
        auxb[pl.ds(0, 16)] = aoff + cnt_i
        auxb[pl.ds(16, 16)] = aoff
        pltpu.sync_copy(auxb, aux_hbm)
        for j in range(2):
            s = (si + 16 * j) * BM
            acc = jnp.zeros((16,), jnp.int32)
            for e in range(E):
                ae = jnp.sum(jnp.where(si == e, aend_al, 0))
                acc = acc + jnp.where(s >= ae, 1, 0)
            teb[pl.ds(16 * j, 16)] = jnp.minimum(acc, E - 1)
        pltpu.sync_copy(teb, te_hbm)

    for j in range(_APW // 16):
        v = idx_all[pl.ds(abase + j * 16, 16)]
        sv[...] = start
        base_g = plsc.load_gather(sv, [v])
        rank = jnp.zeros((16,), jnp.int32)
        for e in range(E):
            m = v == e
            ind = jnp.where(m, 1, 0)
            cs = plsc.cumsum(ind)
            rank = jnp.where(m, cs - ind, rank)
            start = start + jnp.where(si == e, jnp.sum(ind), 0)
        posb[pl.ds(j * 16, 16)] = base_g + rank
        idsb[pl.ds(j * 16, 16)] = wid * (_APW // 2) + ((j * 16 + si) >> 1)

    for j in range(_APW // 32):
        idxe = si * 2 + 32 * j
        p0b[pl.ds(j * 16, 16)] = plsc.load_gather(posb, [idxe])
        p1b[pl.ds(j * 16, 16)] = plsc.load_gather(posb, [idxe + 1])

    pltpu.sync_copy(p0b, p0_hbm.at[pl.ds(wid * _TPW, _TPW)])
    pltpu.sync_copy(p1b, p1_hbm.at[pl.ds(wid * _TPW, _TPW)])
    c0 = pltpu.async_copy(idsb, gid_hbm.at[posb], s0)
    c1 = pltpu.async_copy(wbuf, ws_hbm.at[posb], s1)
    c0.wait()
    c1.wait()


def _sc_route_call(idxr, wr, cnt):
    return pl.kernel(
        _sc_route_body,
        mesh=plsc.VectorSubcoreMesh(core_axis_name="c", subcore_axis_name="s"),
        compiler_params=pltpu.CompilerParams(needs_layout_passes=False),
        out_type=[
            jax.ShapeDtypeStruct((T,), jnp.int32),
            jax.ShapeDtypeStruct((T,), jnp.int32),
            jax.ShapeDtypeStruct((GN,), jnp.int32),
            jax.ShapeDtypeStruct((PAD_N,), jnp.float32),
            jax.ShapeDtypeStruct((32,), jnp.int32),
            jax.ShapeDtypeStruct((32,), jnp.int32),
        ],
        scratch_types=[
            pltpu.VMEM((_A,), jnp.int32),
            pltpu.VMEM((_APW,), jnp.float32),
            pltpu.VMEM((_APW,), jnp.int32),
            pltpu.VMEM((_APW,), jnp.int32),
            pltpu.VMEM((_TPW,), jnp.int32),
            pltpu.VMEM((_TPW,), jnp.int32),
            pltpu.VMEM((16,), jnp.int32),
            pltpu.VMEM((32,), jnp.int32),
            pltpu.VMEM((32,), jnp.int32),
            pltpu.VMEM((16,), jnp.float32),
            pltpu.SemaphoreType.DMA,
            pltpu.SemaphoreType.DMA,
        ],
    )(idxr, wr, cnt)


def _sc_gather_body(
    x_hbm, ids_hbm, aux_hbm, out_hbm, idx0, idx1, idx2, auxv, buf0, buf1,
    g0, g1, wsem
):
    wid = lax.axis_index("s") * 2 + lax.axis_index("c")
    base = wid * _RPW
    pltpu.sync_copy(ids_hbm.at[pl.ds(base, _CH)], idx0)
    pltpu.sync_copy(ids_hbm.at[pl.ds(base + _CH, _CH)], idx1)
    pltpu.sync_copy(ids_hbm.at[pl.ds(base + 2 * _CH, _CH)], idx2)
    pltpu.sync_copy(aux_hbm, auxv)

    si = lax.iota(jnp.int32, 16)
    aend = auxv[pl.ds(0, 16)]
    aoff = auxv[pl.ds(16, 16)]
    bounds = [
        (jnp.sum(jnp.where(si == e, aoff, 0)), jnp.sum(jnp.where(si == e, aend, 0)))
        for e in range(E)
    ]
    for c, ref in ((0, idx0), (1, idx1), (2, idx2)):
        for m in range(_CH // 16):
            p = base + c * _CH + m * 16 + si
            valid = p < 0
            for lo, hi in bounds:
                valid = valid | ((p >= lo) & (p < hi))
            sl = pl.ds(m * 16, 16)
            ref[sl] = jnp.where(valid, ref[sl], p & (T - 1))

    c0 = pltpu.async_copy(x_hbm.at[idx0], buf0, g0)
    c1 = pltpu.async_copy(x_hbm.at[idx1], buf1, g1)
    c0.wait()
    w0 = pltpu.async_copy(buf0, out_hbm.at[pl.ds(base, _CH)], wsem)
    c1.wait()
    w1 = pltpu.async_copy(buf1, out_hbm.at[pl.ds(base + _CH, _CH)], wsem)
    w0.wait()
    c2 = pltpu.async_copy(x_hbm.at[idx2], buf0, g0)
    c2.wait()
    w2 = pltpu.async_copy(buf0, out_hbm.at[pl.ds(base + 2 * _CH, _CH)], wsem)
    w1.wait()
    w2.wait()


def _sc_gather_call(x2, gather_ids, aux):
    return pl.kernel(
        _sc_gather_body,
        mesh=plsc.VectorSubcoreMesh(core_axis_name="c", subcore_axis_name="s"),
        compiler_params=pltpu.CompilerParams(needs_layout_passes=False),
        out_type=jax.ShapeDtypeStruct((GN, D), jnp.float32),
        scratch_types=[
            pltpu.VMEM((_CH,), jnp.int32),
            pltpu.VMEM((_CH,), jnp.int32),
            pltpu.VMEM((_CH,), jnp.int32),
            pltpu.VMEM((32,), jnp.int32),
            pltpu.VMEM((_CH, D), jnp.float32),
            pltpu.VMEM((_CH, D), jnp.float32),
            pltpu.SemaphoreType.DMA,
            pltpu.SemaphoreType.DMA,
            pltpu.SemaphoreType.DMA,
        ],
    )(x2, gather_ids, aux)


def _ffn_body(te_ref, xs_ref, w1_ref, b1_ref, w2_ref, b2_ref, ws_ref, ys_ref):
    del te_ref
    xs = xs_ref[...]
    h = jnp.dot(xs, w1_ref[0], preferred_element_type=jnp.float32)
    h = h + b1_ref[0]
    h = 0.5 * h * (1.0 + lax.erf(h * 0.7071067811865476))
    y = jnp.dot(h, w2_ref[0], preferred_element_type=jnp.float32)
    y = y + b2_ref[0]
    ys_ref[...] = y * ws_ref[...]


def _ffn_call(tile_e, xs, W1, b1, W2, b2, ws):
    grid_spec = pltpu.PrefetchScalarGridSpec(
        num_scalar_prefetch=1,
        grid=(NT,),
        in_specs=[
            pl.BlockSpec((BM, D), lambda i, te: (i, 0)),
            pl.BlockSpec((1, D, F), lambda i, te: (te[i], 0, 0)),
            pl.BlockSpec((1, 1, F), lambda i, te: (te[i], 0, 0)),
            pl.BlockSpec((1, F, D), lambda i, te: (te[i], 0, 0)),
            pl.BlockSpec((1, 1, D), lambda i, te: (te[i], 0, 0)),
            pl.BlockSpec((BM, 1), lambda i, te: (i, 0)),
        ],
        out_specs=pl.BlockSpec((BM, D), lambda i, te: (i, 0)),
    )
    return pl.pallas_call(
        _ffn_body,
        grid_spec=grid_spec,
        out_shape=jax.ShapeDtypeStruct((PAD_N, D), jnp.float32),
    )(tile_e, xs, W1, b1.reshape(E, 1, F), W2, b2.reshape(E, 1, D), ws)


def _sc_combine_body(
    ys_hbm, p0_hbm, p1_hbm, out_hbm, i0_v, i1_v, r0_v, r1_v, sem0, sem1
):
    wid = lax.axis_index("s") * 2 + lax.axis_index("c")
    base = wid * _TPW
    pltpu.sync_copy(p0_hbm.at[pl.ds(base, _TPW)], i0_v)
    pltpu.sync_copy(p1_hbm.at[pl.ds(base, _TPW)], i1_v)
    c0 = pltpu.async_copy(ys_hbm.at[i0_v], r0_v, sem0)
    c1 = pltpu.async_copy(ys_hbm.at[i1_v], r1_v, sem1)
    c0.wait()
    c1.wait()

    def add_row(i, _):
        for j in range(D // _VL):
            sl = pl.ds(j * _VL, _VL)
            r0_v[i, sl] = r0_v[i, sl] + r1_v[i, sl]
        return 0

    lax.fori_loop(0, _TPW, add_row, 0)
    pltpu.sync_copy(r0_v, out_hbm.at[pl.ds(base, _TPW)])


def _sc_combine_call(ys, p0, p1):
    return pl.kernel(
        _sc_combine_body,
        mesh=plsc.VectorSubcoreMesh(core_axis_name="c", subcore_axis_name="s"),
        out_type=jax.ShapeDtypeStruct((T, D), jnp.float32),
        scratch_types=[
            pltpu.VMEM((_TPW,), jnp.int32),
            pltpu.VMEM((_TPW,), jnp.int32),
            pltpu.VMEM((_TPW, D), jnp.float32),
            pltpu.VMEM((_TPW, D), jnp.float32),
            pltpu.SemaphoreType.DMA,
            pltpu.SemaphoreType.DMA,
        ],
    )(ys, p0, p1)


def kernel(x, gate_W, gate_b, W1, b1, W2, b2):
    x2 = x.reshape(T, D)
    logits, idx2, w2, cnt = _gating_call(x2, gate_W, gate_b)
    p0, p1, gather_ids, ws, aux, tile_e = _sc_route_call(
        idx2.reshape(T * K), w2.reshape(T * K), cnt
    )
    xs = _sc_gather_call(x2, gather_ids, aux)
    ys = _ffn_call(tile_e, xs, W1, b1, W2, b2, ws.reshape(PAD_N, 1))
    out = _sc_combine_call(ys, p0, p1)

    return (
        out.reshape(1, T, D),
        logits.reshape(1, T, E),
        idx2.reshape(1, T, K),
        cnt[0, :E],
    )


# speedup vs baseline: 1.0532x; 1.0532x over previous
import functools

import jax
import jax.numpy as jnp
from jax import lax
from jax.experimental import pallas as pl
from jax.experimental.pallas import tpu as pltpu
from jax.experimental.pallas import tpu_sc as plsc

E = 8
K = 2
T = 2048
D = 768
F = 1024
BM = 256
NT = (T * K + E * (BM - 1)) // BM
PAD_N = NT * BM

_NW = 32
GN = 6144
_CH = 96
_RPW = GN // _NW
_TPW = T // _NW
_VL = 16


def _gating_body(x_ref, gw_ref, gb_ref, logits_ref, idx_ref, w_ref, cnt_ref,
                 xp_ref, ph_ref):
    x = x_ref[...]
    pa = lax.bitcast_convert_type(x[:, : D // 2], jnp.uint32)
    pb = lax.bitcast_convert_type(x[:, D // 2 :], jnp.uint32)
    packed = ((pa + 0x8000) & jnp.uint32(0xFFFF0000)) | ((pb + 0x8000) >> 16)
    xp_ref[...] = lax.bitcast_convert_type(packed, jnp.int32)
    logits = jnp.dot(x, gw_ref[...].T, preferred_element_type=jnp.float32)
    logits = logits + gb_ref[...]
    logits_ref[...] = logits

    eio = lax.broadcasted_iota(jnp.int32, (T, E), 1)
    m1 = jnp.max(logits, axis=1, keepdims=True)
    i1 = jnp.min(jnp.where(logits == m1, eio, E), axis=1, keepdims=True)
    masked = jnp.where(eio == i1, -jnp.inf, logits)
    m2 = jnp.max(masked, axis=1, keepdims=True)
    i2 = jnp.min(jnp.where(masked == m2, eio, E), axis=1, keepdims=True)

    t = jnp.exp(m2 - m1)
    s = 1.0 + t
    w1 = 1.0 / s
    w2 = t / s

    idx_ref[...] = jnp.concatenate([i1, i2], axis=1).astype(jnp.int32)
    w_ref[...] = jnp.concatenate([w1, w2], axis=1)
    cnt1 = (eio == i1).astype(jnp.float32) + (eio == i2).astype(jnp.float32)
    cnt = jnp.sum(cnt1, axis=0, keepdims=True)
    cnt_ref[...] = jnp.concatenate([cnt, jnp.zeros((1, 8), jnp.float32)], axis=1)
    bcnt = jnp.sum(cnt1.reshape(32, T // 32, E), axis=1)
    tri = (
        lax.broadcasted_iota(jnp.int32, (32, 32), 0)
        > lax.broadcasted_iota(jnp.int32, (32, 32), 1)
    ).astype(jnp.float32)
    ph = jnp.dot(tri, bcnt, preferred_element_type=jnp.float32)
    ph_ref[...] = jnp.concatenate([ph, jnp.zeros((32, 8), jnp.float32)], axis=1)


def _gating_call(x2, gate_W, gate_b):
    return pl.pallas_call(
        _gating_body,
        out_shape=[
            jax.ShapeDtypeStruct((T, E), jnp.float32),
            jax.ShapeDtypeStruct((T, K), jnp.int32),
            jax.ShapeDtypeStruct((T, K), jnp.float32),
            jax.ShapeDtypeStruct((1, 16), jnp.float32),
            jax.ShapeDtypeStruct((T, D // 2), jnp.int32),
            jax.ShapeDtypeStruct((32, 16), jnp.float32),
        ],
    )(x2, gate_W, gate_b.reshape(1, E))


_A = T * K
_APW = _A // _NW


def _sc_route_body(
    idx_hbm, w_hbm, cnt_hbm, ph_hbm,
    p0_hbm, p1_hbm, gid_hbm, ws_hbm, aux_hbm, te_hbm,
    idx_all, wbuf, posb, idsb, p0b, p1b, sv, auxb, teb, cntv, phv, s0, s1,
):
    wid = lax.axis_index("s") * 2 + lax.axis_index("c")
    abase = wid * _APW
    pltpu.sync_copy(idx_hbm.at[pl.ds(abase, _APW)], idx_all)
    pltpu.sync_copy(w_hbm.at[pl.ds(abase, _APW)], wbuf)
    pltpu.sync_copy(cnt_hbm.at[0], cntv)
    pltpu.sync_copy(ph_hbm.at[wid], phv)

    si = lax.iota(jnp.int32, 16)
    hist_v = phv[...].astype(jnp.int32)
    cnt_i = cntv[...].astype(jnp.int32)
    acnt = ((cnt_i + (BM - 1)) // BM) * BM
    aend_al = plsc.cumsum(acnt)
    aoff = aend_al - acnt
    start = aoff + hist_v

    @pl.when(wid == 0)
    def _aux():
        auxb[pl.ds(0, 16)] = aoff + cnt_i
        auxb[pl.ds(16, 16)] = aoff
        pltpu.sync_copy(auxb, aux_hbm)
        for j in range(2):
            s = (si + 16 * j) * BM
            acc = jnp.zeros((16,), jnp.int32)
            for e in range(E):
                ae = jnp.sum(jnp.where(si == e, aend_al, 0))
                acc = acc + jnp.where(s >= ae, 1, 0)
            teb[pl.ds(16 * j, 16)] = jnp.minimum(acc, E - 1)
        pltpu.sync_copy(teb, te_hbm)

    for j in range(_APW // 16):
        v = idx_all[pl.ds(j * 16, 16)]
        sv[...] = start
        base_g = plsc.load_gather(sv, [v])
        rank = jnp.zeros((16,), jnp.int32)
        for e in range(E):
            m = v == e
            ind = jnp.where(m, 1, 0)
            cs = plsc.cumsum(ind)
            rank = jnp.where(m, cs - ind, rank)
            start = start + jnp.where(si == e, jnp.sum(ind), 0)
        posb[pl.ds(j * 16, 16)] = base_g + rank
        idsb[pl.ds(j * 16, 16)] = wid * (_APW // 2) + ((j * 16 + si) >> 1)

    for j in range(_APW // 32):
        idxe = si * 2 + 32 * j
        p0b[pl.ds(j * 16, 16)] = plsc.load_gather(posb, [idxe])
        p1b[pl.ds(j * 16, 16)] = plsc.load_gather(posb, [idxe + 1])

    pltpu.sync_copy(p0b, p0_hbm.at[pl.ds(wid * _TPW, _TPW)])
    pltpu.sync_copy(p1b, p1_hbm.at[pl.ds(wid * _TPW, _TPW)])
    c0 = pltpu.async_copy(idsb, gid_hbm.at[posb], s0)
    c1 = pltpu.async_copy(wbuf, ws_hbm.at[posb], s1)
    c0.wait()
    c1.wait()


def _sc_route_call(idxr, wr, cnt, ph):
    return pl.kernel(
        _sc_route_body,
        mesh=plsc.VectorSubcoreMesh(core_axis_name="c", subcore_axis_name="s"),
        compiler_params=pltpu.CompilerParams(needs_layout_passes=False),
        out_type=[
            jax.ShapeDtypeStruct((T,), jnp.int32),
            jax.ShapeDtypeStruct((T,), jnp.int32),
            jax.ShapeDtypeStruct((GN,), jnp.int32),
            jax.ShapeDtypeStruct((PAD_N,), jnp.float32),
            jax.ShapeDtypeStruct((32,), jnp.int32),
            jax.ShapeDtypeStruct((32,), jnp.int32),
        ],
        scratch_types=[
            pltpu.VMEM((_APW,), jnp.int32),
            pltpu.VMEM((_APW,), jnp.float32),
            pltpu.VMEM((_APW,), jnp.int32),
            pltpu.VMEM((_APW,), jnp.int32),
            pltpu.VMEM((_TPW,), jnp.int32),
            pltpu.VMEM((_TPW,), jnp.int32),
            pltpu.VMEM((16,), jnp.int32),
            pltpu.VMEM((32,), jnp.int32),
            pltpu.VMEM((32,), jnp.int32),
            pltpu.VMEM((16,), jnp.float32),
            pltpu.VMEM((16,), jnp.float32),
            pltpu.SemaphoreType.DMA,
            pltpu.SemaphoreType.DMA,
        ],
    )(idxr, wr, cnt, ph)


def _sc_gather_body(
    x_hbm, ids_hbm, aux_hbm, out_hbm, idx0, idx1, auxv, buf0, buf1,
    g0, g1, wsem
):
    wid = lax.axis_index("s") * 2 + lax.axis_index("c")
    base = wid * _RPW
    pltpu.sync_copy(ids_hbm.at[pl.ds(base, _CH)], idx0)
    pltpu.sync_copy(ids_hbm.at[pl.ds(base + _CH, _CH)], idx1)
    pltpu.sync_copy(aux_hbm, auxv)

    si = lax.iota(jnp.int32, 16)
    aend = auxv[pl.ds(0, 16)]
    aoff = auxv[pl.ds(16, 16)]
    bounds = [
        (jnp.sum(jnp.where(si == e, aoff, 0)), jnp.sum(jnp.where(si == e, aend, 0)))
        for e in range(E)
    ]
    for c, ref in ((0, idx0), (1, idx1)):
        for m in range(_CH // 16):
            p = base + c * _CH + m * 16 + si
            valid = p < 0
            for lo, hi in bounds:
                valid = valid | ((p >= lo) & (p < hi))
            sl = pl.ds(m * 16, 16)
            ref[sl] = jnp.where(valid, ref[sl], p & (T - 1))

    c0 = pltpu.async_copy(x_hbm.at[idx0], buf0, g0)
    c1 = pltpu.async_copy(x_hbm.at[idx1], buf1, g1)
    c0.wait()
    w0 = pltpu.async_copy(buf0, out_hbm.at[pl.ds(base, _CH)], wsem)
    c1.wait()
    w1 = pltpu.async_copy(buf1, out_hbm.at[pl.ds(base + _CH, _CH)], wsem)
    w0.wait()
    w1.wait()


def _sc_gather_call(xp, gather_ids, aux):
    return pl.kernel(
        _sc_gather_body,
        mesh=plsc.VectorSubcoreMesh(core_axis_name="c", subcore_axis_name="s"),
        compiler_params=pltpu.CompilerParams(needs_layout_passes=False),
        out_type=jax.ShapeDtypeStruct((GN, D // 2), jnp.int32),
        scratch_types=[
            pltpu.VMEM((_CH,), jnp.int32),
            pltpu.VMEM((_CH,), jnp.int32),
            pltpu.VMEM((32,), jnp.int32),
            pltpu.VMEM((_CH, D // 2), jnp.int32),
            pltpu.VMEM((_CH, D // 2), jnp.int32),
            pltpu.SemaphoreType.DMA,
            pltpu.SemaphoreType.DMA,
            pltpu.SemaphoreType.DMA,
        ],
    )(xp, gather_ids, aux)


def _ffn_body(te_ref, xs_ref, w1_ref, b1_ref, w2_ref, b2_ref, ws_ref, ys_ref):
    del te_ref
    xp = lax.bitcast_convert_type(xs_ref[...], jnp.uint32)
    xa = lax.bitcast_convert_type(xp & jnp.uint32(0xFFFF0000), jnp.float32)
    xb = lax.bitcast_convert_type(xp << 16, jnp.float32)
    xs = jnp.concatenate([xa, xb], axis=1).astype(jnp.bfloat16)
    h = jnp.dot(xs, w1_ref[0].astype(jnp.bfloat16),
                preferred_element_type=jnp.float32)
    h = h + b1_ref[0]
    h = 0.5 * h * (1.0 + lax.erf(h * 0.7071067811865476))
    y = jnp.dot(h.astype(jnp.bfloat16), w2_ref[0].astype(jnp.bfloat16),
                preferred_element_type=jnp.float32)
    y = (y + b2_ref[0]) * ws_ref[...]
    pa = lax.bitcast_convert_type(y[:, : D // 2], jnp.uint32)
    pb = lax.bitcast_convert_type(y[:, D // 2 :], jnp.uint32)
    packed = ((pa + 0x8000) & jnp.uint32(0xFFFF0000)) | ((pb + 0x8000) >> 16)
    ys_ref[...] = lax.bitcast_convert_type(packed, jnp.int32)


def _ffn_call(tile_e, xs, W1, b1, W2, b2, ws):
    grid_spec = pltpu.PrefetchScalarGridSpec(
        num_scalar_prefetch=1,
        grid=(NT,),
        in_specs=[
            pl.BlockSpec((BM, D // 2), lambda i, te: (i, 0)),
            pl.BlockSpec((1, D, F), lambda i, te: (te[i], 0, 0)),
            pl.BlockSpec((1, 1, F), lambda i, te: (te[i], 0, 0)),
            pl.BlockSpec((1, F, D), lambda i, te: (te[i], 0, 0)),
            pl.BlockSpec((1, 1, D), lambda i, te: (te[i], 0, 0)),
            pl.BlockSpec((BM, 1), lambda i, te: (i, 0)),
        ],
        out_specs=pl.BlockSpec((BM, D // 2), lambda i, te: (i, 0)),
    )
    return pl.pallas_call(
        _ffn_body,
        grid_spec=grid_spec,
        out_shape=jax.ShapeDtypeStruct((PAD_N, D // 2), jnp.int32),
    )(tile_e, xs, W1, b1.reshape(E, 1, F), W2, b2.reshape(E, 1, D), ws)


def _sc_combine_body(
    ys_hbm, p0_hbm, p1_hbm, out_hbm, i0_v, i1_v, r0_v, r1_v, ob_v, sem0, sem1
):
    wid = lax.axis_index("s") * 2 + lax.axis_index("c")
    base = wid * _TPW
    pltpu.sync_copy(p0_hbm.at[pl.ds(base, _TPW)], i0_v)
    pltpu.sync_copy(p1_hbm.at[pl.ds(base, _TPW)], i1_v)
    c0 = pltpu.async_copy(ys_hbm.at[i0_v], r0_v, sem0)
    c1 = pltpu.async_copy(ys_hbm.at[i1_v], r1_v, sem1)
    c0.wait()
    c1.wait()
    hmask = jnp.int32(-65536)

    def add_row(i, _):
        for j in range(D // 2 // _VL):
            sl = pl.ds(j * _VL, _VL)
            r0 = r0_v[i, sl]
            r1 = r1_v[i, sl]
            a = plsc.bitcast(r0 & hmask, jnp.float32) + plsc.bitcast(
                r1 & hmask, jnp.float32)
            b = plsc.bitcast(r0 << 16, jnp.float32) + plsc.bitcast(
                r1 << 16, jnp.float32)
            ob_v[i, sl] = a
            ob_v[i, pl.ds(D // 2 + j * _VL, _VL)] = b
        return 0

    lax.fori_loop(0, _TPW, add_row, 0)
    pltpu.sync_copy(ob_v, out_hbm.at[pl.ds(base, _TPW)])


def _sc_combine_call(ys, p0, p1):
    return pl.kernel(
        _sc_combine_body,
        mesh=plsc.VectorSubcoreMesh(core_axis_name="c", subcore_axis_name="s"),
        compiler_params=pltpu.CompilerParams(needs_layout_passes=False),
        out_type=jax.ShapeDtypeStruct((T, D), jnp.float32),
        scratch_types=[
            pltpu.VMEM((_TPW,), jnp.int32),
            pltpu.VMEM((_TPW,), jnp.int32),
            pltpu.VMEM((_TPW, D // 2), jnp.int32),
            pltpu.VMEM((_TPW, D // 2), jnp.int32),
            pltpu.VMEM((_TPW, D), jnp.float32),
            pltpu.SemaphoreType.DMA,
            pltpu.SemaphoreType.DMA,
        ],
    )(ys, p0, p1)


def kernel(x, gate_W, gate_b, W1, b1, W2, b2):
    x2 = x.reshape(T, D)
    logits, idx2, w2, cnt, xp, ph = _gating_call(x2, gate_W, gate_b)
    p0, p1, gather_ids, ws, aux, tile_e = _sc_route_call(
        idx2.reshape(T * K), w2.reshape(T * K), cnt, ph
    )
    xs = _sc_gather_call(xp, gather_ids, aux)
    ys = _ffn_call(tile_e, xs, W1, b1, W2, b2, ws.reshape(PAD_N, 1))
    out = _sc_combine_call(ys, p0, p1)

    return (
        out.reshape(1, T, D),
        logits.reshape(1, T, E),
        idx2.reshape(1, T, K),
        cnt[0, :E],
    )


# speedup vs baseline: 1.4609x; 1.3870x over previous
import functools

import jax
import jax.numpy as jnp
from jax import lax
from jax.experimental import pallas as pl
from jax.experimental.pallas import tpu as pltpu
from jax.experimental.pallas import tpu_sc as plsc

E = 8
K = 2
T = 2048
D = 768
F = 1024
BM = 256
NT = (T * K + E * (BM - 1)) // BM
PAD_N = NT * BM

_NW = 32
GN = 6144
_CH = 96
_RPW = GN // _NW
_TPW = T // _NW
_VL = 16


def _gating_body(x_ref, gw_ref, gb_ref, logits_ref, idx_ref, w_ref, cnt_ref,
                 xp_ref, ph_ref):
    x = x_ref[...]
    pa = lax.bitcast_convert_type(x[:, : D // 2], jnp.uint32)
    pb = lax.bitcast_convert_type(x[:, D // 2 :], jnp.uint32)
    packed = ((pa + 0x8000) & jnp.uint32(0xFFFF0000)) | ((pb + 0x8000) >> 16)
    xp_ref[...] = lax.bitcast_convert_type(packed, jnp.int32)
    logits = jnp.dot(x, gw_ref[...].T, preferred_element_type=jnp.float32)
    logits = logits + gb_ref[...]
    logits_ref[...] = logits

    eio = lax.broadcasted_iota(jnp.int32, (T, E), 1)
    m1 = jnp.max(logits, axis=1, keepdims=True)
    i1 = jnp.min(jnp.where(logits == m1, eio, E), axis=1, keepdims=True)
    masked = jnp.where(eio == i1, -jnp.inf, logits)
    m2 = jnp.max(masked, axis=1, keepdims=True)
    i2 = jnp.min(jnp.where(masked == m2, eio, E), axis=1, keepdims=True)

    t = jnp.exp(m2 - m1)
    s = 1.0 + t
    w1 = 1.0 / s
    w2 = t / s

    idx_ref[...] = jnp.concatenate([i1, i2], axis=1).astype(jnp.int32)
    w_ref[...] = jnp.concatenate([w1, w2], axis=1)
    cnt1 = (eio == i1).astype(jnp.float32) + (eio == i2).astype(jnp.float32)
    cnt = jnp.sum(cnt1, axis=0, keepdims=True)
    cnt_ref[...] = jnp.concatenate([cnt, jnp.zeros((1, 8), jnp.float32)], axis=1)
    bcnt = jnp.sum(cnt1.reshape(32, T // 32, E), axis=1)
    tri = (
        lax.broadcasted_iota(jnp.int32, (32, 32), 0)
        > lax.broadcasted_iota(jnp.int32, (32, 32), 1)
    ).astype(jnp.float32)
    ph = jnp.dot(tri, bcnt, preferred_element_type=jnp.float32)
    ph_ref[...] = jnp.concatenate([ph, jnp.zeros((32, 8), jnp.float32)], axis=1)


def _gating_call(x2, gate_W, gate_b):
    return pl.pallas_call(
        _gating_body,
        out_shape=[
            jax.ShapeDtypeStruct((T, E), jnp.float32),
            jax.ShapeDtypeStruct((T, K), jnp.int32),
            jax.ShapeDtypeStruct((T, K), jnp.float32),
            jax.ShapeDtypeStruct((1, 16), jnp.float32),
            jax.ShapeDtypeStruct((T, D // 2), jnp.int32),
            jax.ShapeDtypeStruct((32, 16), jnp.float32),
        ],
    )(x2, gate_W, gate_b.reshape(1, E))


_A = T * K
_APW = _A // _NW


def _sc_route_body(
    idx_hbm, w_hbm, cnt_hbm, ph_hbm,
    p0_hbm, p1_hbm, big_hbm, aux_hbm, te_hbm,
    idx_all, wbuf, posb, valb, p0b, p1b, sv, auxb, teb, cntv, phv, s0, s1,
):
    wid = lax.axis_index("s") * 2 + lax.axis_index("c")
    abase = wid * _APW
    pltpu.sync_copy(idx_hbm.at[pl.ds(abase, _APW)], idx_all)
    pltpu.sync_copy(w_hbm.at[pl.ds(abase, _APW)], wbuf)
    pltpu.sync_copy(cnt_hbm.at[0], cntv)
    pltpu.sync_copy(ph_hbm.at[wid], phv)

    si = lax.iota(jnp.int32, 16)
    hist_v = phv[...].astype(jnp.int32)
    cnt_i = cntv[...].astype(jnp.int32)
    acnt = ((cnt_i + (BM - 1)) // BM) * BM
    aend_al = plsc.cumsum(acnt)
    aoff = aend_al - acnt
    start = aoff + hist_v

    @pl.when(wid == 0)
    def _aux():
        auxb[pl.ds(0, 16)] = aoff + cnt_i
        auxb[pl.ds(16, 16)] = aoff
        pltpu.sync_copy(auxb, aux_hbm)
        for j in range(2):
            s = (si + 16 * j) * BM
            acc = jnp.zeros((16,), jnp.int32)
            for e in range(E):
                ae = jnp.sum(jnp.where(si == e, aend_al, 0))
                acc = acc + jnp.where(s >= ae, 1, 0)
            teb[pl.ds(16 * j, 16)] = jnp.minimum(acc, E - 1)
        pltpu.sync_copy(teb, te_hbm)

    for j in range(_APW // 16):
        v = idx_all[pl.ds(j * 16, 16)]
        sv[...] = start
        base_g = plsc.load_gather(sv, [v])
        rank = jnp.zeros((16,), jnp.int32)
        for e in range(E):
            m = v == e
            ind = jnp.where(m, 1, 0)
            cs = plsc.cumsum(ind)
            rank = jnp.where(m, cs - ind, rank)
            start = start + jnp.where(si == e, jnp.sum(ind), 0)
        posb[pl.ds(j * 16, 16)] = base_g + rank
        tok = wid * (_APW // 2) + ((j * 16 + si) >> 1)
        wv = plsc.bitcast(wbuf[pl.ds(j * 16, 16)], jnp.int32)
        plsc.store_scatter(valb, [si + 16 * j, si * 0], tok)
        plsc.store_scatter(valb, [si + 16 * j, si * 0 + 1], wv)

    for j in range(_APW // 32):
        idxe = si * 2 + 32 * j
        p0b[pl.ds(j * 16, 16)] = plsc.load_gather(posb, [idxe])
        p1b[pl.ds(j * 16, 16)] = plsc.load_gather(posb, [idxe + 1])

    pltpu.sync_copy(p0b, p0_hbm.at[pl.ds(wid * _TPW, _TPW)])
    pltpu.sync_copy(p1b, p1_hbm.at[pl.ds(wid * _TPW, _TPW)])
    pltpu.async_copy(valb, big_hbm.at[posb], s0).wait()


def _sc_route_call(idxr, wr, cnt, ph):
    return pl.kernel(
        _sc_route_body,
        mesh=plsc.VectorSubcoreMesh(core_axis_name="c", subcore_axis_name="s"),
        compiler_params=pltpu.CompilerParams(needs_layout_passes=False),
        out_type=[
            jax.ShapeDtypeStruct((T,), jnp.int32),
            jax.ShapeDtypeStruct((T,), jnp.int32),
            jax.ShapeDtypeStruct((GN, 128), jnp.int32),
            jax.ShapeDtypeStruct((32,), jnp.int32),
            jax.ShapeDtypeStruct((32,), jnp.int32),
        ],
        scratch_types=[
            pltpu.VMEM((_APW,), jnp.int32),
            pltpu.VMEM((_APW,), jnp.float32),
            pltpu.VMEM((_APW,), jnp.int32),
            pltpu.VMEM((_APW, 128), jnp.int32),
            pltpu.VMEM((_TPW,), jnp.int32),
            pltpu.VMEM((_TPW,), jnp.int32),
            pltpu.VMEM((16,), jnp.int32),
            pltpu.VMEM((32,), jnp.int32),
            pltpu.VMEM((32,), jnp.int32),
            pltpu.VMEM((16,), jnp.float32),
            pltpu.VMEM((16,), jnp.float32),
            pltpu.SemaphoreType.DMA,
            pltpu.SemaphoreType.DMA,
        ],
    )(idxr, wr, cnt, ph)


def _sc_gather_body(
    x_hbm, big_hbm, aux_hbm, out_hbm, bigv, idx0, idx1, auxv, buf0, buf1,
    g0, g1, wsem
):
    wid = lax.axis_index("s") * 2 + lax.axis_index("c")
    base = wid * _RPW
    pltpu.sync_copy(big_hbm.at[pl.ds(base, _RPW)], bigv)
    pltpu.sync_copy(aux_hbm, auxv)

    si = lax.iota(jnp.int32, 16)
    aend = auxv[pl.ds(0, 16)]
    aoff = auxv[pl.ds(16, 16)]
    bounds = [
        (jnp.sum(jnp.where(si == e, aoff, 0)), jnp.sum(jnp.where(si == e, aend, 0)))
        for e in range(E)
    ]
    for c, ref in ((0, idx0), (1, idx1)):
        for m in range(_CH // 16):
            ids = plsc.load_gather(bigv, [si + 16 * (c * (_CH // 16) + m), si * 0])
            p = base + c * _CH + m * 16 + si
            valid = p < 0
            for lo, hi in bounds:
                valid = valid | ((p >= lo) & (p < hi))
            ref[pl.ds(m * 16, 16)] = jnp.where(valid, ids, p & (T - 1))

    c0 = pltpu.async_copy(x_hbm.at[idx0], buf0, g0)
    c1 = pltpu.async_copy(x_hbm.at[idx1], buf1, g1)
    c0.wait()
    w0 = pltpu.async_copy(buf0, out_hbm.at[pl.ds(base, _CH)], wsem)
    c1.wait()
    w1 = pltpu.async_copy(buf1, out_hbm.at[pl.ds(base + _CH, _CH)], wsem)
    w0.wait()
    w1.wait()


def _sc_gather_call(xp, big, aux):
    return pl.kernel(
        _sc_gather_body,
        mesh=plsc.VectorSubcoreMesh(core_axis_name="c", subcore_axis_name="s"),
        compiler_params=pltpu.CompilerParams(needs_layout_passes=False),
        out_type=jax.ShapeDtypeStruct((GN, D // 2), jnp.int32),
        scratch_types=[
            pltpu.VMEM((_RPW, 128), jnp.int32),
            pltpu.VMEM((_CH,), jnp.int32),
            pltpu.VMEM((_CH,), jnp.int32),
            pltpu.VMEM((32,), jnp.int32),
            pltpu.VMEM((_CH, D // 2), jnp.int32),
            pltpu.VMEM((_CH, D // 2), jnp.int32),
            pltpu.SemaphoreType.DMA,
            pltpu.SemaphoreType.DMA,
            pltpu.SemaphoreType.DMA,
        ],
    )(xp, big, aux)


def _ffn_body(te_ref, xs_ref, w1_ref, b1_ref, w2_ref, b2_ref, ws_ref, ys_ref):
    del te_ref
    xp = lax.bitcast_convert_type(xs_ref[...], jnp.uint32)
    xa = lax.bitcast_convert_type(xp & jnp.uint32(0xFFFF0000), jnp.float32)
    xb = lax.bitcast_convert_type(xp << 16, jnp.float32)
    xs = jnp.concatenate([xa, xb], axis=1).astype(jnp.bfloat16)
    h = jnp.dot(xs, w1_ref[0].astype(jnp.bfloat16),
                preferred_element_type=jnp.float32)
    h = h + b1_ref[0]
    h = 0.5 * h * (1.0 + lax.erf(h * 0.7071067811865476))
    y = jnp.dot(h.astype(jnp.bfloat16), w2_ref[0].astype(jnp.bfloat16),
                preferred_element_type=jnp.float32)
    w_row = lax.bitcast_convert_type(ws_ref[...][:, 1:2], jnp.float32)
    y = (y + b2_ref[0]) * w_row
    pa = lax.bitcast_convert_type(y[:, : D // 2], jnp.uint32)
    pb = lax.bitcast_convert_type(y[:, D // 2 :], jnp.uint32)
    packed = ((pa + 0x8000) & jnp.uint32(0xFFFF0000)) | ((pb + 0x8000) >> 16)
    ys_ref[...] = lax.bitcast_convert_type(packed, jnp.int32)


def _ffn_call(tile_e, xs, W1, b1, W2, b2, ws):
    grid_spec = pltpu.PrefetchScalarGridSpec(
        num_scalar_prefetch=1,
        grid=(NT,),
        in_specs=[
            pl.BlockSpec((BM, D // 2), lambda i, te: (i, 0)),
            pl.BlockSpec((1, D, F), lambda i, te: (te[i], 0, 0)),
            pl.BlockSpec((1, 1, F), lambda i, te: (te[i], 0, 0)),
            pl.BlockSpec((1, F, D), lambda i, te: (te[i], 0, 0)),
            pl.BlockSpec((1, 1, D), lambda i, te: (te[i], 0, 0)),
            pl.BlockSpec((BM, 128), lambda i, te: (i, 0)),
        ],
        out_specs=pl.BlockSpec((BM, D // 2), lambda i, te: (i, 0)),
    )
    return pl.pallas_call(
        _ffn_body,
        grid_spec=grid_spec,
        out_shape=jax.ShapeDtypeStruct((PAD_N, D // 2), jnp.int32),
    )(tile_e, xs, W1, b1.reshape(E, 1, F), W2, b2.reshape(E, 1, D), ws)


def _sc_combine_body(
    ys_hbm, p0_hbm, p1_hbm, out_hbm, i0_v, i1_v, r0_v, r1_v, ob_v, sem0, sem1
):
    wid = lax.axis_index("s") * 2 + lax.axis_index("c")
    base = wid * _TPW
    pltpu.sync_copy(p0_hbm.at[pl.ds(base, _TPW)], i0_v)
    pltpu.sync_copy(p1_hbm.at[pl.ds(base, _TPW)], i1_v)
    c0 = pltpu.async_copy(ys_hbm.at[i0_v], r0_v, sem0)
    c1 = pltpu.async_copy(ys_hbm.at[i1_v], r1_v, sem1)
    c0.wait()
    c1.wait()
    hmask = jnp.int32(-65536)

    def add_row(i, _):
        for j in range(D // 2 // _VL):
            sl = pl.ds(j * _VL, _VL)
            r0 = r0_v[i, sl]
            r1 = r1_v[i, sl]
            a = plsc.bitcast(r0 & hmask, jnp.float32) + plsc.bitcast(
                r1 & hmask, jnp.float32)
            b = plsc.bitcast(r0 << 16, jnp.float32) + plsc.bitcast(
                r1 << 16, jnp.float32)
            ob_v[i, sl] = a
            ob_v[i, pl.ds(D // 2 + j * _VL, _VL)] = b
        return 0

    lax.fori_loop(0, _TPW, add_row, 0)
    pltpu.sync_copy(ob_v, out_hbm.at[pl.ds(base, _TPW)])


def _sc_combine_call(ys, p0, p1):
    return pl.kernel(
        _sc_combine_body,
        mesh=plsc.VectorSubcoreMesh(core_axis_name="c", subcore_axis_name="s"),
        compiler_params=pltpu.CompilerParams(needs_layout_passes=False),
        out_type=jax.ShapeDtypeStruct((T, D), jnp.float32),
        scratch_types=[
            pltpu.VMEM((_TPW,), jnp.int32),
            pltpu.VMEM((_TPW,), jnp.int32),
            pltpu.VMEM((_TPW, D // 2), jnp.int32),
            pltpu.VMEM((_TPW, D // 2), jnp.int32),
            pltpu.VMEM((_TPW, D), jnp.float32),
            pltpu.SemaphoreType.DMA,
            pltpu.SemaphoreType.DMA,
        ],
    )(ys, p0, p1)


def kernel(x, gate_W, gate_b, W1, b1, W2, b2):
    x2 = x.reshape(T, D)
    logits, idx2, w2, cnt, xp, ph = _gating_call(x2, gate_W, gate_b)
    p0, p1, big, aux, tile_e = _sc_route_call(
        idx2.reshape(T * K), w2.reshape(T * K), cnt, ph
    )
    xs = _sc_gather_call(xp, big, aux)
    ys = _ffn_call(tile_e, xs, W1, b1, W2, b2, big)
    out = _sc_combine_call(ys, p0, p1)

    return (
        out.reshape(1, T, D),
        logits.reshape(1, T, E),
        idx2.reshape(1, T, K),
        cnt[0, :E],
    )


# speedup vs baseline: 1.4822x; 1.0146x over previous
import functools

import jax
import jax.numpy as jnp
from jax import lax
from jax.experimental import pallas as pl
from jax.experimental.pallas import tpu as pltpu
from jax.experimental.pallas import tpu_sc as plsc

E = 8
K = 2
T = 2048
D = 768
F = 1024
BM = 256
NT = (T * K + E * (BM - 1)) // BM
PAD_N = NT * BM

_NW = 32
GN = 6144
_CH = 96
_RPW = GN // _NW
_TPW = T // _NW
_VL = 16


def _gating_body(x_ref, gw_ref, gb_ref, logits_ref, idx_ref, w_ref, cnt_ref,
                 xp_ref, ph_ref):
    x = x_ref[...]
    pa = lax.bitcast_convert_type(x[:, : D // 2], jnp.uint32)
    pb = lax.bitcast_convert_type(x[:, D // 2 :], jnp.uint32)
    packed = ((pa + 0x8000) & jnp.uint32(0xFFFF0000)) | ((pb + 0x8000) >> 16)
    xp_ref[...] = lax.bitcast_convert_type(packed, jnp.int32)
    logits = jnp.dot(x, gw_ref[...].T, preferred_element_type=jnp.float32)
    logits = logits + gb_ref[...]
    logits_ref[...] = logits

    eio = lax.broadcasted_iota(jnp.int32, (T, E), 1)
    m1 = jnp.max(logits, axis=1, keepdims=True)
    i1 = jnp.min(jnp.where(logits == m1, eio, E), axis=1, keepdims=True)
    masked = jnp.where(eio == i1, -jnp.inf, logits)
    m2 = jnp.max(masked, axis=1, keepdims=True)
    i2 = jnp.min(jnp.where(masked == m2, eio, E), axis=1, keepdims=True)

    t = jnp.exp(m2 - m1)
    s = 1.0 + t
    w1 = 1.0 / s
    w2 = t / s

    idx_ref[...] = jnp.concatenate([i1, i2], axis=1).astype(jnp.int32)
    w_ref[...] = jnp.concatenate([w1, w2], axis=1)
    cnt1 = (eio == i1).astype(jnp.float32) + (eio == i2).astype(jnp.float32)
    cnt = jnp.sum(cnt1, axis=0, keepdims=True)
    cnt_ref[...] = jnp.concatenate([cnt, jnp.zeros((1, 8), jnp.float32)], axis=1)
    bcnt = jnp.sum(cnt1.reshape(32, T // 32, E), axis=1)
    tri = (
        lax.broadcasted_iota(jnp.int32, (32, 32), 0)
        > lax.broadcasted_iota(jnp.int32, (32, 32), 1)
    ).astype(jnp.float32)
    ph = jnp.dot(tri, bcnt, preferred_element_type=jnp.float32)
    ph_ref[...] = jnp.concatenate([ph, jnp.zeros((32, 8), jnp.float32)], axis=1)


def _gating_call(x2, gate_W, gate_b):
    return pl.pallas_call(
        _gating_body,
        out_shape=[
            jax.ShapeDtypeStruct((T, E), jnp.float32),
            jax.ShapeDtypeStruct((T, K), jnp.int32),
            jax.ShapeDtypeStruct((T, K), jnp.float32),
            jax.ShapeDtypeStruct((1, 16), jnp.float32),
            jax.ShapeDtypeStruct((T, D // 2), jnp.int32),
            jax.ShapeDtypeStruct((32, 16), jnp.float32),
        ],
    )(x2, gate_W, gate_b.reshape(1, E))


_A = T * K
_APW = _A // _NW


def _sc_route_body(
    idx_hbm, w_hbm, cnt_hbm, ph_hbm,
    p0_hbm, p1_hbm, big_hbm, aux_hbm, te_hbm,
    idx_all, wbuf, posb, valb, p0b, p1b, sv, auxb, teb, cntv, phv, s0, s1,
):
    wid = lax.axis_index("s") * 2 + lax.axis_index("c")
    abase = wid * _APW
    pltpu.sync_copy(idx_hbm.at[pl.ds(abase, _APW)], idx_all)
    pltpu.sync_copy(w_hbm.at[pl.ds(abase, _APW)], wbuf)
    pltpu.sync_copy(cnt_hbm.at[0], cntv)
    pltpu.sync_copy(ph_hbm.at[wid], phv)

    si = lax.iota(jnp.int32, 16)
    hist_v = phv[...].astype(jnp.int32)
    cnt_i = cntv[...].astype(jnp.int32)
    acnt = ((cnt_i + (BM - 1)) // BM) * BM
    aend_al = plsc.cumsum(acnt)
    aoff = aend_al - acnt
    start = aoff + hist_v

    @pl.when(wid == 0)
    def _aux():
        auxb[pl.ds(0, 16)] = aoff + cnt_i
        auxb[pl.ds(16, 16)] = aoff
        pltpu.sync_copy(auxb, aux_hbm)
        for j in range(2):
            s = (si + 16 * j) * BM
            acc = jnp.zeros((16,), jnp.int32)
            for e in range(E):
                ae = jnp.sum(jnp.where(si == e, aend_al, 0))
                acc = acc + jnp.where(s >= ae, 1, 0)
            teb[pl.ds(16 * j, 16)] = acc
        pltpu.sync_copy(teb, te_hbm)

    for j in range(_APW // 16):
        v = idx_all[pl.ds(j * 16, 16)]
        sv[...] = start
        base_g = plsc.load_gather(sv, [v])
        rank = jnp.zeros((16,), jnp.int32)
        for e in range(E):
            m = v == e
            ind = jnp.where(m, 1, 0)
            cs = plsc.cumsum(ind)
            rank = jnp.where(m, cs - ind, rank)
            start = start + jnp.where(si == e, jnp.sum(ind), 0)
        posb[pl.ds(j * 16, 16)] = base_g + rank
        tok = wid * (_APW // 2) + ((j * 16 + si) >> 1)
        wv = plsc.bitcast(wbuf[pl.ds(j * 16, 16)], jnp.int32)
        plsc.store_scatter(valb, [si + 16 * j, si * 0], tok)
        plsc.store_scatter(valb, [si + 16 * j, si * 0 + 1], wv)

    for j in range(_APW // 32):
        idxe = si * 2 + 32 * j
        p0b[pl.ds(j * 16, 16)] = plsc.load_gather(posb, [idxe])
        p1b[pl.ds(j * 16, 16)] = plsc.load_gather(posb, [idxe + 1])

    pltpu.sync_copy(p0b, p0_hbm.at[pl.ds(wid * _TPW, _TPW)])
    pltpu.sync_copy(p1b, p1_hbm.at[pl.ds(wid * _TPW, _TPW)])
    pltpu.async_copy(valb, big_hbm.at[posb], s0).wait()


def _sc_route_call(idxr, wr, cnt, ph):
    return pl.kernel(
        _sc_route_body,
        mesh=plsc.VectorSubcoreMesh(core_axis_name="c", subcore_axis_name="s"),
        compiler_params=pltpu.CompilerParams(needs_layout_passes=False),
        out_type=[
            jax.ShapeDtypeStruct((T,), jnp.int32),
            jax.ShapeDtypeStruct((T,), jnp.int32),
            jax.ShapeDtypeStruct((GN, 128), jnp.int32),
            jax.ShapeDtypeStruct((32,), jnp.int32),
            jax.ShapeDtypeStruct((32,), jnp.int32),
        ],
        scratch_types=[
            pltpu.VMEM((_APW,), jnp.int32),
            pltpu.VMEM((_APW,), jnp.float32),
            pltpu.VMEM((_APW,), jnp.int32),
            pltpu.VMEM((_APW, 128), jnp.int32),
            pltpu.VMEM((_TPW,), jnp.int32),
            pltpu.VMEM((_TPW,), jnp.int32),
            pltpu.VMEM((16,), jnp.int32),
            pltpu.VMEM((32,), jnp.int32),
            pltpu.VMEM((32,), jnp.int32),
            pltpu.VMEM((16,), jnp.float32),
            pltpu.VMEM((16,), jnp.float32),
            pltpu.SemaphoreType.DMA,
            pltpu.SemaphoreType.DMA,
        ],
    )(idxr, wr, cnt, ph)


def _sc_gather_body(
    x_hbm, big_hbm, aux_hbm, out_hbm, bigv, idx0, idx1, auxv, buf0, buf1,
    g0, g1, wsem
):
    wid = lax.axis_index("s") * 2 + lax.axis_index("c")
    base = wid * _RPW
    pltpu.sync_copy(big_hbm.at[pl.ds(base, _RPW)], bigv)
    pltpu.sync_copy(aux_hbm, auxv)

    si = lax.iota(jnp.int32, 16)
    aend = auxv[pl.ds(0, 16)]
    aoff = auxv[pl.ds(16, 16)]
    bounds = [
        (jnp.sum(jnp.where(si == e, aoff, 0)), jnp.sum(jnp.where(si == e, aend, 0)))
        for e in range(E)
    ]
    for c, ref in ((0, idx0), (1, idx1)):
        for m in range(_CH // 16):
            ids = plsc.load_gather(bigv, [si + 16 * (c * (_CH // 16) + m), si * 0])
            p = base + c * _CH + m * 16 + si
            valid = p < 0
            for lo, hi in bounds:
                valid = valid | ((p >= lo) & (p < hi))
            ref[pl.ds(m * 16, 16)] = jnp.where(valid, ids, p & (T - 1))

    c0 = pltpu.async_copy(x_hbm.at[idx0], buf0, g0)
    c1 = pltpu.async_copy(x_hbm.at[idx1], buf1, g1)
    c0.wait()
    w0 = pltpu.async_copy(buf0, out_hbm.at[pl.ds(base, _CH)], wsem)
    c1.wait()
    w1 = pltpu.async_copy(buf1, out_hbm.at[pl.ds(base + _CH, _CH)], wsem)
    w0.wait()
    w1.wait()


def _sc_gather_call(xp, big, aux):
    return pl.kernel(
        _sc_gather_body,
        mesh=plsc.VectorSubcoreMesh(core_axis_name="c", subcore_axis_name="s"),
        compiler_params=pltpu.CompilerParams(needs_layout_passes=False),
        out_type=jax.ShapeDtypeStruct((GN, D // 2), jnp.int32),
        scratch_types=[
            pltpu.VMEM((_RPW, 128), jnp.int32),
            pltpu.VMEM((_CH,), jnp.int32),
            pltpu.VMEM((_CH,), jnp.int32),
            pltpu.VMEM((32,), jnp.int32),
            pltpu.VMEM((_CH, D // 2), jnp.int32),
            pltpu.VMEM((_CH, D // 2), jnp.int32),
            pltpu.SemaphoreType.DMA,
            pltpu.SemaphoreType.DMA,
            pltpu.SemaphoreType.DMA,
        ],
    )(xp, big, aux)


def _ffn_body(te_ref, xs_ref, w1_ref, b1_ref, w2_ref, b2_ref, ws_ref, ys_ref):
    @pl.when(te_ref[pl.program_id(0)] < E)
    def _compute():
        _ffn_tile(xs_ref, w1_ref, b1_ref, w2_ref, b2_ref, ws_ref, ys_ref)


def _ffn_tile(xs_ref, w1_ref, b1_ref, w2_ref, b2_ref, ws_ref, ys_ref):
    xp = lax.bitcast_convert_type(xs_ref[...], jnp.uint32)
    xa = lax.bitcast_convert_type(xp & jnp.uint32(0xFFFF0000), jnp.float32)
    xb = lax.bitcast_convert_type(xp << 16, jnp.float32)
    xs = jnp.concatenate([xa, xb], axis=1).astype(jnp.bfloat16)
    h = jnp.dot(xs, w1_ref[0].astype(jnp.bfloat16),
                preferred_element_type=jnp.float32)
    h = h + b1_ref[0]
    h = 0.5 * h * (1.0 + lax.erf(h * 0.7071067811865476))
    y = jnp.dot(h.astype(jnp.bfloat16), w2_ref[0].astype(jnp.bfloat16),
                preferred_element_type=jnp.float32)
    w_row = lax.bitcast_convert_type(ws_ref[...][:, 1:2], jnp.float32)
    y = (y + b2_ref[0]) * w_row
    pa = lax.bitcast_convert_type(y[:, : D // 2], jnp.uint32)
    pb = lax.bitcast_convert_type(y[:, D // 2 :], jnp.uint32)
    packed = ((pa + 0x8000) & jnp.uint32(0xFFFF0000)) | ((pb + 0x8000) >> 16)
    ys_ref[...] = lax.bitcast_convert_type(packed, jnp.int32)


def _ffn_call(tile_e, xs, W1, b1, W2, b2, ws):
    grid_spec = pltpu.PrefetchScalarGridSpec(
        num_scalar_prefetch=1,
        grid=(NT,),
        in_specs=[
            pl.BlockSpec((BM, D // 2), lambda i, te: (i, 0)),
            pl.BlockSpec((1, D, F), lambda i, te: (jnp.minimum(te[i], E - 1), 0, 0)),
            pl.BlockSpec((1, 1, F), lambda i, te: (jnp.minimum(te[i], E - 1), 0, 0)),
            pl.BlockSpec((1, F, D), lambda i, te: (jnp.minimum(te[i], E - 1), 0, 0)),
            pl.BlockSpec((1, 1, D), lambda i, te: (jnp.minimum(te[i], E - 1), 0, 0)),
            pl.BlockSpec((BM, 128), lambda i, te: (i, 0)),
        ],
        out_specs=pl.BlockSpec((BM, D // 2), lambda i, te: (i, 0)),
    )
    return pl.pallas_call(
        _ffn_body,
        grid_spec=grid_spec,
        out_shape=jax.ShapeDtypeStruct((PAD_N, D // 2), jnp.int32),
    )(tile_e, xs, W1, b1.reshape(E, 1, F), W2, b2.reshape(E, 1, D), ws)


def _sc_combine_body(
    ys_hbm, p0_hbm, p1_hbm, out_hbm, i0_v, i1_v, r0_v, r1_v, ob_v, sem0, sem1
):
    wid = lax.axis_index("s") * 2 + lax.axis_index("c")
    base = wid * _TPW
    pltpu.sync_copy(p0_hbm.at[pl.ds(base, _TPW)], i0_v)
    pltpu.sync_copy(p1_hbm.at[pl.ds(base, _TPW)], i1_v)
    c0 = pltpu.async_copy(ys_hbm.at[i0_v], r0_v, sem0)
    c1 = pltpu.async_copy(ys_hbm.at[i1_v], r1_v, sem1)
    c0.wait()
    c1.wait()
    hmask = jnp.int32(-65536)

    def add_row(i, _):
        for j in range(D // 2 // _VL):
            sl = pl.ds(j * _VL, _VL)
            r0 = r0_v[i, sl]
            r1 = r1_v[i, sl]
            a = plsc.bitcast(r0 & hmask, jnp.float32) + plsc.bitcast(
                r1 & hmask, jnp.float32)
            b = plsc.bitcast(r0 << 16, jnp.float32) + plsc.bitcast(
                r1 << 16, jnp.float32)
            ob_v[i, sl] = a
            ob_v[i, pl.ds(D // 2 + j * _VL, _VL)] = b
        return 0

    lax.fori_loop(0, _TPW, add_row, 0)
    pltpu.sync_copy(ob_v, out_hbm.at[pl.ds(base, _TPW)])


def _sc_combine_call(ys, p0, p1):
    return pl.kernel(
        _sc_combine_body,
        mesh=plsc.VectorSubcoreMesh(core_axis_name="c", subcore_axis_name="s"),
        compiler_params=pltpu.CompilerParams(needs_layout_passes=False),
        out_type=jax.ShapeDtypeStruct((T, D), jnp.float32),
        scratch_types=[
            pltpu.VMEM((_TPW,), jnp.int32),
            pltpu.VMEM((_TPW,), jnp.int32),
            pltpu.VMEM((_TPW, D // 2), jnp.int32),
            pltpu.VMEM((_TPW, D // 2), jnp.int32),
            pltpu.VMEM((_TPW, D), jnp.float32),
            pltpu.SemaphoreType.DMA,
            pltpu.SemaphoreType.DMA,
        ],
    )(ys, p0, p1)


def kernel(x, gate_W, gate_b, W1, b1, W2, b2):
    x2 = x.reshape(T, D)
    logits, idx2, w2, cnt, xp, ph = _gating_call(x2, gate_W, gate_b)
    p0, p1, big, aux, tile_e = _sc_route_call(
        idx2.reshape(T * K), w2.reshape(T * K), cnt, ph
    )
    xs = _sc_gather_call(xp, big, aux)
    ys = _ffn_call(tile_e, xs, W1, b1, W2, b2, big)
    out = _sc_combine_call(ys, p0, p1)

    return (
        out.reshape(1, T, D),
        logits.reshape(1, T, E),
        idx2.reshape(1, T, K),
        cnt[0, :E],
    )
